# Initial kernel scaffold; baseline (speedup 1.0000x reference)
import jax
import jax.numpy as jnp
from jax.experimental import pallas as pl


def kernel(dt, event_types, log_alpha, log_beta):
    raise NotImplementedError("write your pallas kernel here")


# speedup vs baseline: 6.3715x; 6.3715x over previous
import functools

import jax
import jax.numpy as jnp
from jax import lax
from jax.experimental import pallas as pl
from jax.experimental.pallas import tpu as pltpu
from jax.experimental.pallas import tpu_sc as plsc

NC = 2
NS = 16
NW = NC * NS
LANES = 16

K = 26
TPAD = 688
C = 800


def _sc_call(m_total, dtf, etf, la, lb):
    ew = m_total // NW
    blocks = ew // C
    mesh = plsc.VectorSubcoreMesh(
        core_axis_name="c", subcore_axis_name="s", num_cores=NC, num_subcores=NS
    )

    @functools.partial(
        pl.kernel,
        mesh=mesh,
        out_type=jax.ShapeDtypeStruct((m_total * K,), jnp.float32),
        scratch_types=[
            pltpu.VMEM((TPAD,), jnp.float32),
            pltpu.VMEM((TPAD,), jnp.float32),
            pltpu.VMEM((C,), jnp.float32),
            pltpu.VMEM((C,), jnp.int32),
            pltpu.VMEM((C * K,), jnp.float32),
        ],
        compiler_params=pltpu.CompilerParams(needs_layout_passes=False),
    )
    def run(dt_hbm, et_hbm, la_hbm, lb_hbm, out_hbm, tbl_a, tbl_nb, dt_v, et_v, out_v):
        wid = lax.axis_index("s") * NC + lax.axis_index("c")
        pltpu.sync_copy(la_hbm, tbl_a)
        pltpu.sync_copy(lb_hbm, tbl_nb)
        for t in range(TPAD // LANES):
            s = pl.ds(t * LANES, LANES)
            tbl_a[s] = jnp.exp(tbl_a[s])
            tbl_nb[s] = -jnp.exp(tbl_nb[s])

        iota = lax.iota(jnp.int32, LANES)
        iota_k = iota * K
        base_w = wid * ew

        def block(g, carry):
            eb = base_w + g * C
            pltpu.sync_copy(dt_hbm.at[pl.ds(eb, C)], dt_v)
            pltpu.sync_copy(et_hbm.at[pl.ds(eb, C)], et_v)

            @plsc.parallel_loop(0, C // LANES)
            def vstep(v):
                dtv = dt_v[pl.ds(v * LANES, LANES)]
                etv = et_v[pl.ds(v * LANES, LANES)]
                ti = etv * K
                ob = iota_k + v * (LANES * K)

                @plsc.parallel_loop(0, K, unroll=2)
                def kstep(k):
                    a = plsc.load_gather(tbl_a, [ti + k])
                    nb = plsc.load_gather(tbl_nb, [ti + k])
                    r = a * jnp.exp(nb * dtv)
                    plsc.store_scatter(out_v, [ob + k], r)

            pltpu.sync_copy(out_v, out_hbm.at[pl.ds(eb * K, C * K)])
            return carry

        lax.fori_loop(0, blocks, block, 0)

    return run(dtf, etf, la, lb)


def kernel(dt, event_types, log_alpha, log_beta):
    shape = dt.shape
    m_total = dt.size
    dtf = dt.reshape(-1)
    etf = event_types.reshape(-1).astype(jnp.int32)
    pad = TPAD - K * K
    la = jnp.pad(log_alpha.reshape(-1), (0, pad))
    lb = jnp.pad(log_beta.reshape(-1), (0, pad))
    out = _sc_call(m_total, dtf, etf, la, lb)
    return out.reshape(*shape, K)
